# baseline (speedup 1.0000x reference)
import functools

import jax
import jax.numpy as jnp
from jax import lax
from jax.experimental import pallas as pl
from jax.experimental.pallas import tpu as pltpu
from jax.experimental.pallas import tpu_sc as plsc

NC, NS = 2, 16
NW = NC * NS
B, S, D = 16384, 20, 50
G = 16
TROW = 128
DPAD = 64
SENT_PER_CHUNK = 4
TOK_PER_CHUNK = SENT_PER_CHUNK * S
SENT_PER_W = B // NW
CHUNKS_PER_W = SENT_PER_W // SENT_PER_CHUNK
NBUF = 2
INV_SQRT_S = float(1.0 / (S ** 0.5))

_mesh = plsc.VectorSubcoreMesh(
    core_axis_name="c", subcore_axis_name="s", num_cores=NC, num_subcores=NS)


@functools.partial(
    pl.kernel,
    out_type=jax.ShapeDtypeStruct((B, DPAD), jnp.float32),
    mesh=_mesh,
    scratch_types=[
        pltpu.VMEM((CHUNKS_PER_W, TOK_PER_CHUNK), jnp.int32),
        pltpu.VMEM((TOK_PER_CHUNK, TROW), jnp.float32),
        pltpu.VMEM((TOK_PER_CHUNK, TROW), jnp.float32),
        pltpu.VMEM((SENT_PER_W, DPAD), jnp.float32),
        pltpu.SemaphoreType.DMA,
        pltpu.SemaphoreType.DMA,
    ],
    compiler_params=pltpu.CompilerParams(use_tc_tiling_on_sc=True,
                                         needs_layout_passes=False),
)
def _sc_embed(x_hbm, tpad_hbm, out_hbm, ids_v, rows0, rows1,
              sent_v, sem0, sem1):
    wid = lax.axis_index("s") * NC + lax.axis_index("c")

    pltpu.sync_copy(x_hbm.at[wid], ids_v)

    rows = (rows0, rows1)
    sems = (sem0, sem1)

    def copy(i, b):
        return pltpu.make_async_copy(
            tpad_hbm.at[ids_v.at[i]], rows[b], sems[b])

    for b in range(NBUF):
        copy(b, b).start()

    def outer(o, carry):
        for b in range(NBUF):
            i = o * NBUF + b
            copy(i, b).wait()
            for t in range(SENT_PER_CHUNK):
                row0 = t * S
                for c in range(4):
                    acc = rows[b][row0, pl.ds(c * G, G)]
                    for s2 in range(1, S):
                        acc = acc + rows[b][row0 + s2, pl.ds(c * G, G)]
                    sent_v[i * SENT_PER_CHUNK + t, pl.ds(c * G, G)] = (
                        acc * INV_SQRT_S)
            nxt = i + NBUF
            @pl.when(nxt < CHUNKS_PER_W)
            def _():
                copy(nxt, b).start()
        return carry

    lax.fori_loop(0, CHUNKS_PER_W // NBUF, outer, 0)
    pltpu.sync_copy(sent_v, out_hbm.at[pl.ds(wid * SENT_PER_W, SENT_PER_W)])


def _mlp_body(sent_ref, w1_ref, b1_ref, w2_ref, b2_ref, out_ref):
    s = sent_ref[...]
    h = jnp.dot(s, w1_ref[...], preferred_element_type=jnp.float32)
    h = jnp.maximum(h + b1_ref[...], 0.0)
    out_ref[...] = jnp.sum(h * w2_ref[...], axis=1, keepdims=True) + b2_ref[...]


def kernel(x, table, W1, b1, W2, b2):
    x3 = x.reshape(NW, CHUNKS_PER_W, TOK_PER_CHUNK).astype(jnp.int32)
    tpad = jnp.pad(table.astype(jnp.float32), ((0, 0), (0, TROW - D)))
    sent = _sc_embed(x3, tpad)

    w1p = jnp.zeros((DPAD, 16), jnp.float32).at[:D].set(W1.astype(jnp.float32))
    BLK = 2048
    out = pl.pallas_call(
        _mlp_body,
        grid=(B // BLK,),
        in_specs=[
            pl.BlockSpec((BLK, DPAD), lambda i: (i, 0)),
            pl.BlockSpec((DPAD, 16), lambda i: (0, 0)),
            pl.BlockSpec((1, 16), lambda i: (0, 0)),
            pl.BlockSpec((1, 16), lambda i: (0, 0)),
            pl.BlockSpec((1, 1), lambda i: (0, 0)),
        ],
        out_specs=pl.BlockSpec((BLK, 1), lambda i: (i, 0)),
        out_shape=jax.ShapeDtypeStruct((B, 1), jnp.float32),
    )(sent, w1p, b1.reshape(1, 16).astype(jnp.float32),
      W2.reshape(1, 16).astype(jnp.float32),
      b2.reshape(1, 1).astype(jnp.float32))
    return out


# speedup vs baseline: 1.3292x; 1.3292x over previous
import functools

import jax
import jax.numpy as jnp
from jax import lax
from jax.experimental import pallas as pl
from jax.experimental.pallas import tpu as pltpu
from jax.experimental.pallas import tpu_sc as plsc

NC, NS = 2, 16
NW = NC * NS
B, S, D = 16384, 20, 50
G = 16
TROW = 128
DPAD = 64
SENT_PER_CHUNK = 4
TOK_PER_CHUNK = SENT_PER_CHUNK * S
SENT_PER_W = B // NW
CHUNKS_PER_W = SENT_PER_W // SENT_PER_CHUNK
NBUF = 2
INV_SQRT_S = float(1.0 / (S ** 0.5))

_mesh = plsc.VectorSubcoreMesh(
    core_axis_name="c", subcore_axis_name="s", num_cores=NC, num_subcores=NS)


@functools.partial(
    pl.kernel,
    out_type=jax.ShapeDtypeStruct((B, DPAD), jnp.float32),
    mesh=_mesh,
    scratch_types=[
        pltpu.VMEM((CHUNKS_PER_W, TOK_PER_CHUNK), jnp.int32),
        pltpu.VMEM((CHUNKS_PER_W, TOK_PER_CHUNK), jnp.int32),
        pltpu.VMEM((TOK_PER_CHUNK, TROW), jnp.float32),
        pltpu.VMEM((TOK_PER_CHUNK, TROW), jnp.float32),
        pltpu.VMEM((SENT_PER_W, DPAD), jnp.float32),
        pltpu.SemaphoreType.DMA,
        pltpu.SemaphoreType.DMA,
    ],
    compiler_params=pltpu.CompilerParams(use_tc_tiling_on_sc=False,
                                         needs_layout_passes=False),
)
def _sc_embed(xg_hbm, xo_hbm, tpack_hbm, out_hbm, idg_v, ido_v,
              rows0, rows1, sent_v, sem0, sem1):
    wid = lax.axis_index("s") * NC + lax.axis_index("c")

    pltpu.sync_copy(xg_hbm.at[wid], idg_v)
    pltpu.sync_copy(xo_hbm.at[wid], ido_v)

    rows = (rows0, rows1)
    sems = (sem0, sem1)

    def copy(i, b):
        return pltpu.make_async_copy(
            tpack_hbm.at[idg_v.at[i]], rows[b], sems[b])

    for b in range(NBUF):
        copy(b, b).start()

    def outer(o, carry):
        for b in range(NBUF):
            i = o * NBUF + b
            copy(i, b).wait()
            for t in range(SENT_PER_CHUNK):
                row0 = t * S
                offa = ido_v[i, pl.ds(row0, G)]
                offb = ido_v[i, pl.ds(row0 + S - G, G)]
                for c in range(4):
                    def ld(s2):
                        off = offa[s2] if s2 < G else offb[s2 - (S - G)]
                        return rows[b][row0 + s2, pl.ds(off + c * G, G)]
                    acc = ld(0)
                    for s2 in range(1, S):
                        acc = acc + ld(s2)
                    sent_v[i * SENT_PER_CHUNK + t, pl.ds(c * G, G)] = (
                        acc * INV_SQRT_S)
            nxt = i + NBUF
            @pl.when(nxt < CHUNKS_PER_W)
            def _():
                copy(nxt, b).start()
        return carry

    lax.fori_loop(0, CHUNKS_PER_W // NBUF, outer, 0)
    pltpu.sync_copy(sent_v, out_hbm.at[pl.ds(wid * SENT_PER_W, SENT_PER_W)])


def _repack_body(in_ref, out_ref):
    x = in_ref[...]
    half = x.shape[0] // 2
    x3 = x.reshape(half, 2, D)
    z = jnp.zeros((half, DPAD - D), jnp.float32)
    out_ref[...] = jnp.concatenate(
        [x3[:, 0, :], z, x3[:, 1, :], z], axis=1)


def _repack(table):
    rblk = 4000
    grid = table.shape[0] // rblk
    return pl.pallas_call(
        _repack_body,
        grid=(grid,),
        in_specs=[pl.BlockSpec((rblk, D), lambda i: (i, 0))],
        out_specs=pl.BlockSpec((rblk // 2, TROW), lambda i: (i, 0)),
        out_shape=jax.ShapeDtypeStruct((table.shape[0] // 2, TROW),
                                       jnp.float32),
    )(table)


def _mlp_body(sent_ref, w1_ref, b1_ref, w2_ref, b2_ref, out_ref):
    s = sent_ref[...]
    h = jnp.dot(s, w1_ref[...], preferred_element_type=jnp.float32)
    h = jnp.maximum(h + b1_ref[...], 0.0)
    out_ref[...] = jnp.sum(h * w2_ref[...], axis=1, keepdims=True) + b2_ref[...]


def kernel(x, table, W1, b1, W2, b2):
    x32 = x.astype(jnp.int32)
    xg = (x32 >> 1).reshape(NW, CHUNKS_PER_W, TOK_PER_CHUNK)
    xo = ((x32 & 1) * DPAD).reshape(NW, CHUNKS_PER_W, TOK_PER_CHUNK)
    tpack = _repack(table.astype(jnp.float32))
    sent = _sc_embed(xg, xo, tpack)

    w1p = jnp.zeros((DPAD, 16), jnp.float32).at[:D].set(W1.astype(jnp.float32))
    BLK = 2048
    out = pl.pallas_call(
        _mlp_body,
        grid=(B // BLK,),
        in_specs=[
            pl.BlockSpec((BLK, DPAD), lambda i: (i, 0)),
            pl.BlockSpec((DPAD, 16), lambda i: (0, 0)),
            pl.BlockSpec((1, 16), lambda i: (0, 0)),
            pl.BlockSpec((1, 16), lambda i: (0, 0)),
            pl.BlockSpec((1, 1), lambda i: (0, 0)),
        ],
        out_specs=pl.BlockSpec((BLK, 1), lambda i: (i, 0)),
        out_shape=jax.ShapeDtypeStruct((B, 1), jnp.float32),
    )(sent, w1p, b1.reshape(1, 16).astype(jnp.float32),
      W2.reshape(1, 16).astype(jnp.float32),
      b2.reshape(1, 1).astype(jnp.float32))
    return out


# speedup vs baseline: 1.5447x; 1.1622x over previous
import functools

import jax
import jax.numpy as jnp
from jax import lax
from jax.experimental import pallas as pl
from jax.experimental.pallas import tpu as pltpu
from jax.experimental.pallas import tpu_sc as plsc

NC, NS = 2, 16
NW = NC * NS
B, S, D = 16384, 20, 50
G = 16
TROW = 128
DPAD = 64
SENT_PER_CHUNK = 4
TOK_PER_CHUNK = SENT_PER_CHUNK * S
SENT_PER_W = B // NW
CHUNKS_PER_W = SENT_PER_W // SENT_PER_CHUNK
NBUF = 2
INV_SQRT_S = float(1.0 / (S ** 0.5))

_mesh = plsc.VectorSubcoreMesh(
    core_axis_name="c", subcore_axis_name="s", num_cores=NC, num_subcores=NS)


@functools.partial(
    pl.kernel,
    out_type=jax.ShapeDtypeStruct((B, DPAD), jnp.float32),
    mesh=_mesh,
    scratch_types=[
        pltpu.VMEM((CHUNKS_PER_W, TOK_PER_CHUNK), jnp.int32),
        pltpu.VMEM((CHUNKS_PER_W, TOK_PER_CHUNK), jnp.int32),
        pltpu.VMEM((TOK_PER_CHUNK, TROW), jnp.float32),
        pltpu.VMEM((TOK_PER_CHUNK, TROW), jnp.float32),
        pltpu.VMEM((SENT_PER_W, DPAD), jnp.float32),
        pltpu.SemaphoreType.DMA,
        pltpu.SemaphoreType.DMA,
    ],
    compiler_params=pltpu.CompilerParams(use_tc_tiling_on_sc=False,
                                         needs_layout_passes=False),
)
def _sc_embed(xg_hbm, xo_hbm, tpack_hbm, out_hbm, idg_v, ido_v,
              rows0, rows1, sent_v, sem0, sem1):
    wid = lax.axis_index("s") * NC + lax.axis_index("c")

    pltpu.sync_copy(xg_hbm.at[wid], idg_v)
    pltpu.sync_copy(xo_hbm.at[wid], ido_v)

    rows = (rows0, rows1)
    sems = (sem0, sem1)

    def copy(i, b):
        return pltpu.make_async_copy(
            tpack_hbm.at[idg_v.at[i]], rows[b], sems[b])

    for b in range(NBUF):
        copy(b, b).start()

    def outer(o, carry):
        for b in range(NBUF):
            i = o * NBUF + b
            copy(i, b).wait()
            for t in range(SENT_PER_CHUNK):
                row0 = t * S
                offa = ido_v[i, pl.ds(row0, G)]
                offb = ido_v[i, pl.ds(row0 + S - G, G)]
                for c in range(4):
                    def ld(s2):
                        off = offa[s2] if s2 < G else offb[s2 - (S - G)]
                        return rows[b][row0 + s2, pl.ds(off + c * G, G)]
                    acc = ld(0)
                    for s2 in range(1, S):
                        acc = acc + ld(s2)
                    sent_v[i * SENT_PER_CHUNK + t, pl.ds(c * G, G)] = (
                        acc * INV_SQRT_S)
            nxt = i + NBUF
            @pl.when(nxt < CHUNKS_PER_W)
            def _():
                copy(nxt, b).start()
        return carry

    lax.fori_loop(0, CHUNKS_PER_W // NBUF, outer, 0)
    pltpu.sync_copy(sent_v, out_hbm.at[pl.ds(wid * SENT_PER_W, SENT_PER_W)])


def _repack_body(a_ref, b_ref, out_ref):
    a = a_ref[...]
    b = b_ref[...]
    z = jnp.zeros((a.shape[0], DPAD - D), jnp.float32)
    out_ref[...] = jnp.concatenate([a, z, b, z], axis=1)


def _repack(table):
    half = table.shape[0] // 2
    rblk = 2000
    grid = half // rblk
    return pl.pallas_call(
        _repack_body,
        grid=(grid,),
        in_specs=[
            pl.BlockSpec((rblk, D), lambda i: (i, 0)),
            pl.BlockSpec((rblk, D), lambda i, g=grid: (i + g, 0)),
        ],
        out_specs=pl.BlockSpec((rblk, TROW), lambda i: (i, 0)),
        out_shape=jax.ShapeDtypeStruct((half, TROW), jnp.float32),
    )(table, table)


def _mlp_body(sent_ref, w1_ref, b1_ref, w2_ref, b2_ref, out_ref):
    s = sent_ref[...]
    h = jnp.dot(s, w1_ref[...], preferred_element_type=jnp.float32)
    h = jnp.maximum(h + b1_ref[...], 0.0)
    out_ref[...] = jnp.sum(h * w2_ref[...], axis=1, keepdims=True) + b2_ref[...]


def kernel(x, table, W1, b1, W2, b2):
    x32 = x.astype(jnp.int32)
    half = table.shape[0] // 2
    hi = x32 >= half
    xg = jnp.where(hi, x32 - half, x32).reshape(
        NW, CHUNKS_PER_W, TOK_PER_CHUNK)
    xo = jnp.where(hi, DPAD, 0).astype(jnp.int32).reshape(
        NW, CHUNKS_PER_W, TOK_PER_CHUNK)
    tpack = _repack(table.astype(jnp.float32))
    sent = _sc_embed(xg, xo, tpack)

    w1p = jnp.zeros((DPAD, 16), jnp.float32).at[:D].set(W1.astype(jnp.float32))
    BLK = 2048
    out = pl.pallas_call(
        _mlp_body,
        grid=(B // BLK,),
        in_specs=[
            pl.BlockSpec((BLK, DPAD), lambda i: (i, 0)),
            pl.BlockSpec((DPAD, 16), lambda i: (0, 0)),
            pl.BlockSpec((1, 16), lambda i: (0, 0)),
            pl.BlockSpec((1, 16), lambda i: (0, 0)),
            pl.BlockSpec((1, 1), lambda i: (0, 0)),
        ],
        out_specs=pl.BlockSpec((BLK, 1), lambda i: (i, 0)),
        out_shape=jax.ShapeDtypeStruct((B, 1), jnp.float32),
    )(sent, w1p, b1.reshape(1, 16).astype(jnp.float32),
      W2.reshape(1, 16).astype(jnp.float32),
      b2.reshape(1, 1).astype(jnp.float32))
    return out
